# baseline (speedup 1.0000x reference)
import functools

import jax
import jax.numpy as jnp
from jax import lax
from jax.experimental import pallas as pl
from jax.experimental.pallas import tpu as pltpu
from jax.experimental.pallas import tpu_sc as plsc

B, S, D = 4, 2048, 1024
_NC = 2
_NW = 32
_SPW = S // _NW
_NSETS = 3
_GROWS = 8
_PEROWS = 16
_GPC = _PEROWS // _GROWS
_NG = _SPW // _GROWS

_XBUF = pltpu.VMEM((_GROWS, D), jnp.float32)
_DSEM = pltpu.SemaphoreType.DMA


@functools.partial(
    pl.kernel,
    mesh=plsc.VectorSubcoreMesh(core_axis_name="c", subcore_axis_name="s"),
    out_type=jax.ShapeDtypeStruct((B, S, D), jnp.float32),
    scratch_types=(
        [pltpu.VMEM((_PEROWS, D), jnp.float32)]
        + [_XBUF] * (_NSETS * B)
        + [_DSEM] * (_NSETS * B)
        + [_DSEM] * (_NSETS * B)
    ),
)
def _sc_add(x_hbm, pe_hbm, out_hbm, pe_v, *bufs_and_sems):
    nb = _NSETS * B
    xbufs = bufs_and_sems[:nb]
    sins = bufs_and_sems[nb : 2 * nb]
    souts = bufs_and_sems[2 * nb : 3 * nb]
    sets = [
        (xbufs[s * B : (s + 1) * B], sins[s * B : (s + 1) * B],
         souts[s * B : (s + 1) * B])
        for s in range(_NSETS)
    ]
    wid = lax.axis_index("s") * _NC + lax.axis_index("c")
    base = wid * _SPW
    in_dma = [None] * _NSETS
    out_dma = [None] * _NSETS

    def issue_in(k, si):
        xb, sin, _ = sets[si]
        row = base + k * _GROWS
        return [
            pltpu.async_copy(x_hbm.at[b, pl.ds(row, _GROWS)], xb[b], sin[b])
            for b in range(B)
        ]

    in_dma[0] = issue_in(0, 0)
    pltpu.sync_copy(pe_hbm.at[pl.ds(base, _PEROWS)], pe_v)
    for k in range(_NG):
        si = k % _NSETS
        xb, _, sout = sets[si]
        if k > 0 and k % _GPC == 0:
            pltpu.sync_copy(
                pe_hbm.at[pl.ds(base + (k // _GPC) * _PEROWS, _PEROWS)], pe_v
            )
        for d in in_dma[si]:
            d.wait()
        if k + 1 < _NG:
            nxt = (k + 1) % _NSETS
            if out_dma[nxt] is not None:
                for d in out_dma[nxt]:
                    d.wait()
            in_dma[nxt] = issue_in(k + 1, nxt)

        prow = (k % _GPC) * _GROWS

        @plsc.parallel_loop(0, _GROWS * D, step=16, unroll=4)
        def add_body(i, xb=xb, prow=prow):
            r = i >> 10
            c = pl.multiple_of(i & (D - 1), 16)
            pv = pe_v[prow + r, pl.ds(c, 16)]
            for b in range(B):
                xb[b][r, pl.ds(c, 16)] = xb[b][r, pl.ds(c, 16)] + pv

        row = base + k * _GROWS
        out_dma[si] = [
            pltpu.async_copy(xb[b], out_hbm.at[b, pl.ds(row, _GROWS)], sout[b])
            for b in range(B)
        ]
    for dl in out_dma:
        if dl is not None:
            for d in dl:
                d.wait()


def kernel(x, pe):
    return _sc_add(x, pe[:S])


# speedup vs baseline: 1.0214x; 1.0214x over previous
import functools

import jax
import jax.numpy as jnp
from jax import lax
from jax.experimental import pallas as pl
from jax.experimental.pallas import tpu as pltpu
from jax.experimental.pallas import tpu_sc as plsc

B, S, D = 4, 2048, 1024
_NC = 2
_NW = 32
_SPW = S // _NW
_NSETS = 2
_GROWS = 8
_PEROWS = 32
_GPC = _PEROWS // _GROWS
_NG = _SPW // _GROWS

_XBUF = pltpu.VMEM((_GROWS, D), jnp.float32)
_DSEM = pltpu.SemaphoreType.DMA


@functools.partial(
    pl.kernel,
    mesh=plsc.VectorSubcoreMesh(core_axis_name="c", subcore_axis_name="s"),
    out_type=jax.ShapeDtypeStruct((B, S, D), jnp.float32),
    scratch_types=(
        [pltpu.VMEM((_PEROWS, D), jnp.float32)]
        + [_XBUF] * (_NSETS * B)
        + [_DSEM] * (_NSETS * B)
        + [_DSEM] * (_NSETS * B)
    ),
)
def _sc_add(x_hbm, pe_hbm, out_hbm, pe_v, *bufs_and_sems):
    nb = _NSETS * B
    xbufs = bufs_and_sems[:nb]
    sins = bufs_and_sems[nb : 2 * nb]
    souts = bufs_and_sems[2 * nb : 3 * nb]
    sets = [
        (xbufs[s * B : (s + 1) * B], sins[s * B : (s + 1) * B],
         souts[s * B : (s + 1) * B])
        for s in range(_NSETS)
    ]
    wid = lax.axis_index("s") * _NC + lax.axis_index("c")
    base = wid * _SPW
    in_dma = [None] * _NSETS
    out_dma = [None] * _NSETS

    def issue_in(k, si):
        xb, sin, _ = sets[si]
        row = base + k * _GROWS
        return [
            pltpu.async_copy(x_hbm.at[b, pl.ds(row, _GROWS)], xb[b], sin[b])
            for b in range(B)
        ]

    in_dma[0] = issue_in(0, 0)
    pltpu.sync_copy(pe_hbm.at[pl.ds(base, _PEROWS)], pe_v)
    for k in range(_NG):
        si = k % _NSETS
        xb, _, sout = sets[si]
        if k > 0 and k % _GPC == 0:
            pltpu.sync_copy(
                pe_hbm.at[pl.ds(base + (k // _GPC) * _PEROWS, _PEROWS)], pe_v
            )
        for d in in_dma[si]:
            d.wait()
        if k + 1 < _NG:
            nxt = (k + 1) % _NSETS
            if out_dma[nxt] is not None:
                for d in out_dma[nxt]:
                    d.wait()
            in_dma[nxt] = issue_in(k + 1, nxt)

        prow = (k % _GPC) * _GROWS

        @plsc.parallel_loop(0, _GROWS * D, step=16, unroll=4)
        def add_body(i, xb=xb, prow=prow):
            r = i >> 10
            c = pl.multiple_of(i & (D - 1), 16)
            pv = pe_v[prow + r, pl.ds(c, 16)]
            for b in range(B):
                xb[b][r, pl.ds(c, 16)] = xb[b][r, pl.ds(c, 16)] + pv

        row = base + k * _GROWS
        out_dma[si] = [
            pltpu.async_copy(xb[b], out_hbm.at[b, pl.ds(row, _GROWS)], sout[b])
            for b in range(B)
        ]
    for dl in out_dma:
        if dl is not None:
            for d in dl:
                d.wait()


def kernel(x, pe):
    return _sc_add(x, pe[:S])
